# Initial kernel scaffold; baseline (speedup 1.0000x reference)
import jax
import jax.numpy as jnp
from jax.experimental import pallas as pl


def kernel(x, y, z, Wg, bg, Wmu, bmu, Wlv, blv):
    raise NotImplementedError("write your pallas kernel here")


# speedup vs baseline: 1.6789x; 1.6789x over previous
import jax
import jax.numpy as jnp
from jax.experimental import pallas as pl

_N, _D, _E = 2048, 768, 8


def _gate_kernel(x_ref, wg_ref, bg_ref, g4_ref, g1_ref, gloss_ref):
    x = x_ref[:]
    logits = jnp.dot(x, wg_ref[:], preferred_element_type=jnp.float32) + bg_ref[:]
    m = jnp.max(logits, axis=-1, keepdims=True)
    ex = jnp.exp(logits - m)
    gs = ex / jnp.sum(ex, axis=-1, keepdims=True)
    e_iota = jax.lax.broadcasted_iota(jnp.int32, gs.shape, 1)
    rank = jnp.zeros(gs.shape, dtype=jnp.int32)
    for j in range(_E):
        gj = gs[:, j:j + 1]
        hit = (gj > gs) | ((gj == gs) & (j < e_iota))
        rank = rank + hit.astype(jnp.int32)
    mask4 = (rank < 4).astype(jnp.float32)
    mask1 = (rank < 1).astype(jnp.float32)
    g4_ref[:] = gs * mask4
    g1_ref[:] = gs * mask1
    density = jnp.mean(mask4, axis=0, keepdims=True)
    proxy = jnp.mean(gs, axis=0, keepdims=True)
    gloss_ref[:] = jnp.reshape(jnp.mean(density * proxy) * float(_E * _E), (1, 1))


def _x_kernel(x_ref, wmu_ref, bmu_ref, wlv_ref, blv_ref, g4_ref,
              ox_ref, lacc_ref):
    e = pl.program_id(0)
    x = x_ref[:]
    a = jnp.dot(x, wmu_ref[0], preferred_element_type=jnp.float32)
    b = jnp.dot(x, wlv_ref[0], preferred_element_type=jnp.float32)
    mu = a + bmu_ref[0] + x
    lv = b + blv_ref[0]
    exl = jnp.exp(lv)
    elem = (mu * mu + exl - lv - 1.0) * 0.5
    sel = (jax.lax.broadcasted_iota(jnp.int32, (_N, _E), 1) == e).astype(jnp.float32)
    g4 = jnp.sum(g4_ref[:] * sel, axis=1, keepdims=True)
    kl_sum = jnp.sum(elem)
    u = jnp.sum(exl, axis=1, keepdims=True)
    u_sum = jnp.sum(g4 * u)
    contrib = kl_sum / float(_N * _E) + u_sum / float(_N)

    @pl.when(e == 0)
    def _():
        ox_ref[:] = g4 * mu
        lacc_ref[:] = jnp.reshape(contrib, (1, 1))

    @pl.when(e != 0)
    def _():
        ox_ref[:] += g4 * mu
        lacc_ref[:] += jnp.reshape(contrib, (1, 1))


def _yz_kernel(y_ref, z_ref, wmu_ref, bmu_ref, g1_ref, oy_ref, oz_ref):
    e = pl.program_id(0)
    wmu = wmu_ref[0]
    sel = (jax.lax.broadcasted_iota(jnp.int32, (_N, _E), 1) == e).astype(jnp.float32)
    g1 = jnp.sum(g1_ref[:] * sel, axis=1, keepdims=True)
    muy = jnp.dot(y_ref[:], wmu, preferred_element_type=jnp.float32) + bmu_ref[0] + y_ref[:]
    muz = jnp.dot(z_ref[:], wmu, preferred_element_type=jnp.float32) + bmu_ref[0] + z_ref[:]

    @pl.when(e == 0)
    def _():
        oy_ref[:] = g1 * muy
        oz_ref[:] = g1 * muz

    @pl.when(e != 0)
    def _():
        oy_ref[:] += g1 * muy
        oz_ref[:] += g1 * muz


def kernel(x, y, z, Wg, bg, Wmu, bmu, Wlv, blv):
    f32 = jnp.float32
    g4, g1, gloss = pl.pallas_call(
        _gate_kernel,
        out_shape=(
            jax.ShapeDtypeStruct((_N, _E), f32),
            jax.ShapeDtypeStruct((_N, _E), f32),
            jax.ShapeDtypeStruct((1, 1), f32),
        ),
    )(x, Wg, bg.reshape(1, _E))

    bmu3 = bmu.reshape(_E, 1, _D)
    blv3 = blv.reshape(_E, 1, _D)

    ox, lacc = pl.pallas_call(
        _x_kernel,
        grid=(_E,),
        in_specs=[
            pl.BlockSpec((_N, _D), lambda e: (0, 0)),
            pl.BlockSpec((1, _D, _D), lambda e: (e, 0, 0)),
            pl.BlockSpec((1, 1, _D), lambda e: (e, 0, 0)),
            pl.BlockSpec((1, _D, _D), lambda e: (e, 0, 0)),
            pl.BlockSpec((1, 1, _D), lambda e: (e, 0, 0)),
            pl.BlockSpec((_N, _E), lambda e: (0, 0)),
        ],
        out_specs=(
            pl.BlockSpec((_N, _D), lambda e: (0, 0)),
            pl.BlockSpec((1, 1), lambda e: (0, 0)),
        ),
        out_shape=(
            jax.ShapeDtypeStruct((_N, _D), f32),
            jax.ShapeDtypeStruct((1, 1), f32),
        ),
    )(x, Wmu, bmu3, Wlv, blv3, g4)

    oy, oz = pl.pallas_call(
        _yz_kernel,
        grid=(_E,),
        in_specs=[
            pl.BlockSpec((_N, _D), lambda e: (0, 0)),
            pl.BlockSpec((_N, _D), lambda e: (0, 0)),
            pl.BlockSpec((1, _D, _D), lambda e: (e, 0, 0)),
            pl.BlockSpec((1, 1, _D), lambda e: (e, 0, 0)),
            pl.BlockSpec((_N, _E), lambda e: (0, 0)),
        ],
        out_specs=(
            pl.BlockSpec((_N, _D), lambda e: (0, 0)),
            pl.BlockSpec((_N, _D), lambda e: (0, 0)),
        ),
        out_shape=(
            jax.ShapeDtypeStruct((_N, _D), f32),
            jax.ShapeDtypeStruct((_N, _D), f32),
        ),
    )(y, z, Wmu, bmu3, g1)

    loss = gloss[0, 0] + lacc[0, 0]
    return ox, oy, oz, loss
